# Initial kernel scaffold; baseline (speedup 1.0000x reference)
import jax
import jax.numpy as jnp
from jax.experimental import pallas as pl


def kernel(logits, labels):
    raise NotImplementedError("write your pallas kernel here")


# speedup vs baseline: 5.3753x; 5.3753x over previous
import functools

import jax
import jax.numpy as jnp
import numpy as np
from jax.experimental import pallas as pl

_B, _N = 4096, 8192
_K_OUT = 101
_D0 = 64
_W0 = 128
_KCAP = 128
_ROWS = 8


def _batcher_pk(n):
    steps = []
    p = 1
    while p < n:
        k = p
        while k >= 1:
            steps.append((p, k))
            k //= 2
        p *= 2
    return steps


_LEAF_STEPS = _batcher_pk(_D0)


def _depth_iota(shape):
    return jax.lax.broadcasted_iota(jnp.int32, shape, dimension=1)


def _roll(x, shift):
    shift %= x.shape[1]
    if shift == 0:
        return x
    return jnp.concatenate([x[:, -shift:, :], x[:, :-shift, :]], axis=1)


def _leaf_masks(shape, p, k):
    a = _depth_iota(shape)
    if k == p:
        top = (a % (2 * k)) < k
    else:
        top = ((a % (2 * k)) >= k) & ((a % (2 * p)) < 2 * p - k)
    if k == p:
        bot = (a % (2 * k)) >= k
    else:
        ak = a - k
        bot = (a >= k) & ((ak % (2 * k)) >= k) & ((ak % (2 * p)) < 2 * p - k)
    return top, bot


def _leaf_sort(x, dir_b):
    shape = (1, x.shape[1], 1)
    for p, k in _LEAF_STEPS:
        top_b, bot_b = _leaf_masks(shape, p, k)
        up = _roll(x, -k)
        down = _roll(x, k)
        partner = jnp.where(top_b, up, down)
        hi = jnp.maximum(x, partner)
        lo = jnp.minimum(x, partner)
        want_max = (dir_b & top_b) | (~dir_b & bot_b)
        new = jnp.where(want_max, hi, lo)
        x = jnp.where(top_b | bot_b, new, x)
    return x


def _bitonic_merge(z, dir_b):
    depth = z.shape[1]
    idx = _depth_iota((1, depth, 1))
    d = depth // 2
    while d >= 1:
        top_b = (idx % (2 * d)) < d
        up = _roll(z, -d)
        down = _roll(z, d)
        partner = jnp.where(top_b, up, down)
        hi = jnp.maximum(z, partner)
        lo = jnp.minimum(z, partner)
        want_max = (dir_b & top_b) | (~dir_b & ~top_b)
        z = jnp.where(want_max, hi, lo)
        d //= 2
    return z


def _dirs(w):
    lane = jax.lax.broadcasted_iota(jnp.int32, (1, 1, w), dimension=2)
    if w == 1:
        return lane < 1
    return lane < (w // 2)


def _block_kernel(logits_ref, labels_ref, out_logits_ref, out_labels_ref):
    lg = logits_ref[...]
    lb = labels_ref[...]
    rows = lg.shape[0]

    pos_logit = jnp.sum(lg * lb, axis=1, keepdims=True)
    pos_label = jnp.sum(lb, axis=1, keepdims=True)

    neg = jnp.float32(-jnp.inf)
    ml = jnp.where(lb != 0, neg, lg)
    y = ml.reshape(rows, _D0, _W0)
    y = _leaf_sort(y, _dirs(_W0))

    w = _W0
    while w > 1:
        a = y[:, :, : w // 2]
        b = y[:, :, w // 2 :]
        if y.shape[1] < _KCAP:
            z = jnp.concatenate([a, b], axis=1)
        else:
            z = jnp.maximum(a, b)
        w //= 2
        y = _bitonic_merge(z, _dirs(w))

    top = y.reshape(rows, _KCAP)
    out_logits_ref[...] = jnp.concatenate(
        [pos_logit, top[:, : _K_OUT - 1]], axis=1)
    out_labels_ref[...] = jnp.concatenate(
        [pos_label, jnp.zeros((rows, _K_OUT - 1), jnp.float32)], axis=1)


@jax.jit
def kernel(logits, labels):
    grid = (_B // _ROWS,)
    in_spec = pl.BlockSpec((_ROWS, _N), lambda i: (i, 0))
    out_spec = pl.BlockSpec((_ROWS, _K_OUT), lambda i: (i, 0))
    out_logits, out_labels = pl.pallas_call(
        _block_kernel,
        grid=grid,
        in_specs=[in_spec, in_spec],
        out_specs=[out_spec, out_spec],
        out_shape=[
            jax.ShapeDtypeStruct((_B, _K_OUT), jnp.float32),
            jax.ShapeDtypeStruct((_B, _K_OUT), jnp.float32),
        ],
    )(logits, labels)
    return (out_logits, out_labels)


# speedup vs baseline: 37.8039x; 7.0329x over previous
import jax
import jax.numpy as jnp
from jax.experimental import pallas as pl

_B, _N = 4096, 8192
_K_OUT = 101
_D0 = 64
_LANES = 128


def _batcher_pairs(n):
    pairs = []
    p = 1
    while p < n:
        k = p
        while k >= 1:
            for j in range(k % p, n - k, 2 * k):
                for i in range(0, min(k, n - j - k)):
                    if (i + j) // (p * 2) == (i + j + k) // (p * 2):
                        pairs.append((i + j, i + j + k))
            k //= 2
        p *= 2
    return pairs


_LEAF_PAIRS = _batcher_pairs(_D0)


def _split_half(a):
    if a.shape[0] > 1:
        h = a.shape[0] // 2
        return a[:h], a[h:]
    s = a.shape[1] // 2
    return a[:, :s], a[:, s:]


def _rounds(zs, split):
    n = len(zs)
    if split:
        zs = [list(_split_half(a)) for a in zs]
    else:
        zs = [[a] for a in zs]
    d = n // 2
    while d >= 1:
        for i in range(n):
            if (i % (2 * d)) < d:
                j = i + d
                for pidx in range(len(zs[i])):
                    a, b = zs[i][pidx], zs[j][pidx]
                    hi = jnp.maximum(a, b)
                    lo = jnp.minimum(a, b)
                    if split and pidx == 1:
                        zs[i][pidx], zs[j][pidx] = lo, hi
                    else:
                        zs[i][pidx], zs[j][pidx] = hi, lo
        d //= 2
    return zs


def _block_kernel(lgT_ref, lbT_ref, out_lgT_ref, out_lbT_ref):
    lgT = lgT_ref[...]
    lbT = lbT_ref[...]

    pos_row = jnp.sum(lgT * lbT, axis=0, keepdims=True)
    lab_row = jnp.sum(lbT, axis=0, keepdims=True)

    ml = jnp.where(lbT != 0, jnp.float32(-jnp.inf), lgT)

    ps, qs = [], []
    for d in range(_D0):
        e = ml[d * 128 : (d + 1) * 128, :].reshape(16, 8, _LANES)
        ps.append(e[:8])
        qs.append(e[8:])
    for i, j in _LEAF_PAIRS:
        ps[i], ps[j] = jnp.maximum(ps[i], ps[j]), jnp.minimum(ps[i], ps[j])
        qs[i], qs[j] = jnp.minimum(qs[i], qs[j]), jnp.maximum(qs[i], qs[j])

    zs = _rounds(ps + qs, split=True)

    while True:
        merged = [jnp.maximum(e[0], e[1]) for e in zs]
        last = merged[0].shape == (1, 1, _LANES)
        zs = _rounds(merged, split=not last)
        if last:
            break

    top_rows = [zs[j][0].reshape(1, _LANES) for j in range(127)]
    out_lgT_ref[...] = jnp.concatenate([pos_row] + top_rows, axis=0)
    zero_rows = jnp.zeros((127, _LANES), jnp.float32)
    out_lbT_ref[...] = jnp.concatenate([lab_row, zero_rows], axis=0)


@jax.jit
def kernel(logits, labels):
    lgT = logits.T
    lbT = labels.T
    grid = (_B // _LANES,)
    in_spec = pl.BlockSpec((_N, _LANES), lambda i: (0, i))
    out_spec = pl.BlockSpec((128, _LANES), lambda i: (0, i))
    out_lgT, out_lbT = pl.pallas_call(
        _block_kernel,
        grid=grid,
        in_specs=[in_spec, in_spec],
        out_specs=[out_spec, out_spec],
        out_shape=[
            jax.ShapeDtypeStruct((128, _B), jnp.float32),
            jax.ShapeDtypeStruct((128, _B), jnp.float32),
        ],
    )(lgT, lbT)
    return (out_lgT.T[:, :_K_OUT], out_lbT.T[:, :_K_OUT])
